# Initial kernel scaffold; baseline (speedup 1.0000x reference)
import jax
import jax.numpy as jnp
from jax.experimental import pallas as pl


def kernel(x, table):
    raise NotImplementedError("write your pallas kernel here")


# speedup vs baseline: 5.9823x; 5.9823x over previous
import functools
import jax
import jax.numpy as jnp
from jax import lax
from jax.experimental import pallas as pl
from jax.experimental.pallas import tpu as pltpu
from jax.experimental.pallas import tpu_sc as plsc

BATCH = 16384
SEQ = 50
EMBED_DIM = 64
NUM_INDICES = BATCH * SEQ
NUM_WORKERS = 32
PER_WORKER = NUM_INDICES // NUM_WORKERS
CHUNK = 800
NUM_CHUNKS = PER_WORKER // CHUNK


def kernel(x, table):
    idx = x.reshape(NUM_INDICES)

    mesh = plsc.VectorSubcoreMesh(core_axis_name="c", subcore_axis_name="s")

    @jax.jit
    def run(table, idx):
        @functools.partial(
            pl.kernel,
            mesh=mesh,
            out_type=jax.ShapeDtypeStruct((NUM_INDICES, EMBED_DIM), table.dtype),
            scratch_types=[
                pltpu.VMEM((CHUNK,), jnp.int32),
                pltpu.VMEM((CHUNK, EMBED_DIM), jnp.float32),
                pltpu.SemaphoreType.DMA,
            ],
            compiler_params=pltpu.CompilerParams(use_tc_tiling_on_sc=False),
        )
        def sc_gather(table_hbm, idx_hbm, out_hbm, idx_v, rows_v, sem):
            wid = lax.axis_index("s") * 2 + lax.axis_index("c")
            base = wid * PER_WORKER

            @pl.loop(0, NUM_CHUNKS)
            def _(c):
                off = base + c * CHUNK
                pltpu.sync_copy(idx_hbm.at[pl.ds(off, CHUNK)], idx_v)
                pltpu.async_copy(table_hbm.at[idx_v], rows_v, sem).wait()
                pltpu.sync_copy(rows_v, out_hbm.at[pl.ds(off, CHUNK)])

        return sc_gather(table, idx)

    out = run(table, idx)
    return out.reshape(BATCH, SEQ, EMBED_DIM)


# speedup vs baseline: 6.2066x; 1.0375x over previous
import jax
import jax.numpy as jnp
from jax.experimental import pallas as pl
from jax.experimental.pallas import tpu as pltpu
from jax.experimental.pallas import tpu_sc as plsc

BATCH = 16384
SEQ = 50
EMBED_DIM = 64
NUM_INDICES = BATCH * SEQ
WINDOW = 512


def kernel(x, table):
    idx = x.reshape(1, NUM_INDICES)

    mesh = plsc.VectorSubcoreMesh(core_axis_name="c", subcore_axis_name="s")

    @jax.jit
    def run(table, idx):
        @pl.kernel(
            out_type=jax.ShapeDtypeStruct((NUM_INDICES, EMBED_DIM), table.dtype),
            mesh=mesh,
            compiler_params=pltpu.CompilerParams(use_tc_tiling_on_sc=False),
        )
        def sc_gather(table_hbm, idx_hbm, out_hbm):
            def body(idx_vmem, out_vmem):
                pltpu.sync_copy(table_hbm.at[idx_vmem.at[0]], out_vmem)

            pltpu.emit_pipeline(
                body,
                grid=(NUM_INDICES // WINDOW,),
                in_specs=[
                    pl.BlockSpec((1, WINDOW), index_map=lambda i: (0, i))
                ],
                out_specs=[
                    pl.BlockSpec((WINDOW, EMBED_DIM), index_map=lambda i: (i, 0))
                ],
                core_axis_name=("c", "s"),
                dimension_semantics=(pltpu.PARALLEL,),
            )(idx_hbm, out_hbm)

        return sc_gather(table, idx)

    out = run(table, idx)
    return out.reshape(BATCH, SEQ, EMBED_DIM)


# speedup vs baseline: 6.2097x; 1.0005x over previous
import jax
import jax.numpy as jnp
from jax.experimental import pallas as pl
from jax.experimental.pallas import tpu as pltpu
from jax.experimental.pallas import tpu_sc as plsc

BATCH = 16384
SEQ = 50
EMBED_DIM = 64
NUM_INDICES = BATCH * SEQ
WINDOW = 800


def kernel(x, table):
    idx = x.reshape(1, NUM_INDICES)

    mesh = plsc.VectorSubcoreMesh(core_axis_name="c", subcore_axis_name="s")

    @jax.jit
    def run(table, idx):
        @pl.kernel(
            out_type=jax.ShapeDtypeStruct((NUM_INDICES, EMBED_DIM), table.dtype),
            mesh=mesh,
            compiler_params=pltpu.CompilerParams(use_tc_tiling_on_sc=False),
        )
        def sc_gather(table_hbm, idx_hbm, out_hbm):
            def body(idx_vmem, out_vmem):
                pltpu.sync_copy(table_hbm.at[idx_vmem.at[0]], out_vmem)

            pltpu.emit_pipeline(
                body,
                grid=(NUM_INDICES // WINDOW,),
                in_specs=[
                    pl.BlockSpec((1, WINDOW), index_map=lambda i: (0, i))
                ],
                out_specs=[
                    pl.BlockSpec((WINDOW, EMBED_DIM), index_map=lambda i: (i, 0))
                ],
                core_axis_name=("c", "s"),
                dimension_semantics=(pltpu.PARALLEL,),
            )(idx_hbm, out_hbm)

        return sc_gather(table, idx)

    out = run(table, idx)
    return out.reshape(BATCH, SEQ, EMBED_DIM)


# speedup vs baseline: 6.2157x; 1.0010x over previous
import functools
import jax
import jax.numpy as jnp
from jax.experimental import pallas as pl
from jax.experimental.pallas import tpu as pltpu
from jax.experimental.pallas import tpu_sc as plsc

BATCH = 16384
SEQ = 50
EMBED_DIM = 64
NUM_INDICES = BATCH * SEQ
WINDOW = 800
NSTREAM = 4
SUB = WINDOW // NSTREAM


def kernel(x, table):
    idx = x.reshape(1, NUM_INDICES)

    mesh = plsc.VectorSubcoreMesh(core_axis_name="c", subcore_axis_name="s")

    @jax.jit
    def run(table, idx):
        @functools.partial(
            pl.kernel,
            out_type=jax.ShapeDtypeStruct((NUM_INDICES, EMBED_DIM), table.dtype),
            mesh=mesh,
            scratch_types=[pltpu.SemaphoreType.DMA],
            compiler_params=pltpu.CompilerParams(use_tc_tiling_on_sc=False),
        )
        def sc_gather(table_hbm, idx_hbm, out_hbm, sem):
            def body(idx_vmem, out_vmem):
                handles = [
                    pltpu.async_copy(
                        table_hbm.at[idx_vmem.at[0, pl.ds(q * SUB, SUB)]],
                        out_vmem.at[pl.ds(q * SUB, SUB)],
                        sem,
                    )
                    for q in range(NSTREAM)
                ]
                for h in handles:
                    h.wait()

            pltpu.emit_pipeline(
                body,
                grid=(NUM_INDICES // WINDOW,),
                in_specs=[
                    pl.BlockSpec((1, WINDOW), index_map=lambda i: (0, i))
                ],
                out_specs=[
                    pl.BlockSpec((WINDOW, EMBED_DIM), index_map=lambda i: (i, 0))
                ],
                core_axis_name=("c", "s"),
                dimension_semantics=(pltpu.PARALLEL,),
            )(idx_hbm, out_hbm)

        return sc_gather(table, idx)

    out = run(table, idx)
    return out.reshape(BATCH, SEQ, EMBED_DIM)
